# baseline (speedup 1.0000x reference)
import functools

import jax
import jax.numpy as jnp
from jax import lax
from jax.experimental import pallas as pl
from jax.experimental.pallas import tpu as pltpu
from jax.experimental.pallas import tpu_sc as plsc

_NC = 2
_NS = 16
_NW = _NC * _NS
_L = 16
_C = 125


def _stage1_body(nodes_ref, enc_W_ref, enc_b_ref, emb_W_ref, emb_b_ref,
                 vodes_ref, embed_ref):
    x = nodes_ref[...]
    vodes_ref[...] = jnp.maximum(
        jnp.dot(x, enc_W_ref[...], preferred_element_type=jnp.float32)
        + enc_b_ref[...], 0.0)
    embed_ref[...] = (
        jnp.dot(x, emb_W_ref[...], preferred_element_type=jnp.float32)
        + emb_b_ref[...])


def _stage1(nodes, enc_W, enc_b, emb_W, emb_b):
    n, d = nodes.shape
    rows = 1000 if n % 1000 == 0 else n
    grid = n // rows
    de = enc_W.shape[1]
    dm = emb_W.shape[1]
    return pl.pallas_call(
        _stage1_body,
        grid=(grid,),
        in_specs=[
            pl.BlockSpec((rows, d), lambda i: (i, 0)),
            pl.BlockSpec((d, de), lambda i: (0, 0)),
            pl.BlockSpec((1, de), lambda i: (0, 0)),
            pl.BlockSpec((d, dm), lambda i: (0, 0)),
            pl.BlockSpec((1, dm), lambda i: (0, 0)),
        ],
        out_specs=[
            pl.BlockSpec((rows, de), lambda i: (i, 0)),
            pl.BlockSpec((rows, dm), lambda i: (i, 0)),
        ],
        out_shape=[
            jax.ShapeDtypeStruct((n, de), jnp.float32),
            jax.ShapeDtypeStruct((n, dm), jnp.float32),
        ],
    )(nodes, enc_W, enc_b.reshape(1, de), emb_W, emb_b.reshape(1, dm))


def _sc_body(n, de, n_chunks, vodes_hbm, s2d_hbm, r2d_hbm, out_hbm,
             sbuf, rbuf, rows_v0, rows_v1, zbuf, acc_sh, vodes_sh,
             sem0, sem1):
    cid = lax.axis_index("c")
    sid = lax.axis_index("s")
    wid = sid * _NC + cid
    rows_lo = (n // _NS) // 8 * 8
    tail = n - _NS * rows_lo
    base = sid * rows_lo

    pltpu.sync_copy(s2d_hbm.at[pl.ds(wid * n_chunks, n_chunks)], sbuf)
    pltpu.sync_copy(r2d_hbm.at[pl.ds(wid * n_chunks, n_chunks)], rbuf)

    pltpu.sync_copy(vodes_hbm.at[pl.ds(base, rows_lo)],
                    vodes_sh.at[pl.ds(base, rows_lo)])

    @pl.when(sid == _NS - 1)
    def _():
        pltpu.sync_copy(vodes_hbm.at[pl.ds(_NS * rows_lo, tail)],
                        vodes_sh.at[pl.ds(_NS * rows_lo, tail)])

    def zero_body(i, _):
        zbuf[i, pl.ds(0, _L)] = jnp.zeros((_L,), jnp.float32)
        zbuf[i, pl.ds(_L, _L)] = jnp.zeros((_L,), jnp.float32)
        return 0

    lax.fori_loop(0, rows_lo, zero_body, 0)
    pltpu.sync_copy(zbuf, acc_sh.at[pl.ds(base, rows_lo)])

    @pl.when(sid == _NS - 1)
    def _():
        pltpu.sync_copy(zbuf.at[pl.ds(0, tail)],
                        acc_sh.at[pl.ds(_NS * rows_lo, tail)])

    plsc.subcore_barrier()

    n_pairs = n_chunks // 2

    def chunk_body(k, _):
        j0 = 2 * k
        j1 = j0 + 1
        pltpu.make_async_copy(vodes_sh.at[sbuf.at[j0]], rows_v0, sem0).wait()
        pltpu.async_copy(vodes_hbm.at[sbuf.at[j1]], rows_v1, sem1)
        pltpu.sync_copy(rows_v0, acc_sh.at[rbuf.at[j0]], add=True)
        pltpu.make_async_copy(vodes_hbm.at[sbuf.at[j1]], rows_v1, sem1).wait()
        pltpu.async_copy(vodes_sh.at[sbuf.at[j0 + 2]], rows_v0, sem0)
        pltpu.sync_copy(rows_v1, acc_sh.at[rbuf.at[j1]], add=True)
        return 0

    pltpu.async_copy(vodes_sh.at[sbuf.at[0]], rows_v0, sem0)
    lax.fori_loop(0, n_pairs - 1, chunk_body, 0)
    jl = n_chunks - 2
    pltpu.make_async_copy(vodes_sh.at[sbuf.at[jl]], rows_v0, sem0).wait()
    pltpu.async_copy(vodes_hbm.at[sbuf.at[jl + 1]], rows_v1, sem1)
    pltpu.sync_copy(rows_v0, acc_sh.at[rbuf.at[jl]], add=True)
    pltpu.make_async_copy(vodes_hbm.at[sbuf.at[jl + 1]], rows_v1, sem1).wait()
    pltpu.sync_copy(rows_v1, acc_sh.at[rbuf.at[jl + 1]], add=True)
    plsc.subcore_barrier()
    pltpu.sync_copy(acc_sh.at[pl.ds(base, rows_lo)],
                    out_hbm.at[cid, pl.ds(base, rows_lo)])

    @pl.when(sid == _NS - 1)
    def _():
        pltpu.sync_copy(acc_sh.at[pl.ds(_NS * rows_lo, tail)],
                        out_hbm.at[cid, pl.ds(_NS * rows_lo, tail)])


def _stage2(vodes, senders, receivers):
    n, de = vodes.shape
    e = senders.shape[0]
    n_chunks = e // (_NW * _C)
    s2d = senders.reshape(_NW * n_chunks, _C)
    r2d = receivers.reshape(_NW * n_chunks, _C)
    mesh = plsc.VectorSubcoreMesh(core_axis_name="c", subcore_axis_name="s")
    return pl.kernel(
        functools.partial(_sc_body, n, de, n_chunks),
        out_type=jax.ShapeDtypeStruct((_NC, n, de), jnp.float32),
        mesh=mesh,
        compiler_params=pltpu.CompilerParams(needs_layout_passes=False,
                                             use_tc_tiling_on_sc=False),
        scratch_types=[
            pltpu.VMEM((n_chunks, _C), jnp.int32),
            pltpu.VMEM((n_chunks, _C), jnp.int32),
            pltpu.VMEM((_C, de), jnp.float32),
            pltpu.VMEM((_C, de), jnp.float32),
            pltpu.VMEM(((n // _NS) // 8 * 8, de), jnp.float32),
            pltpu.VMEM_SHARED((n, de), jnp.float32),
            pltpu.VMEM_SHARED((n, de), jnp.float32),
            pltpu.SemaphoreType.DMA,
            pltpu.SemaphoreType.DMA,
        ],
    )(vodes, s2d, r2d)


def _stage3_body(vodes_ref, partials_ref, embed_ref, lastcol_ref, infw_ref,
                 infb_ref, w1_ref, b1_ref, w2_ref, b2_ref, wy_ref, by_ref,
                 wx_ref, bx_ref, logits_ref, value_ref):
    msg = partials_ref[0] + partials_ref[1]
    vodc = jnp.concatenate([vodes_ref[...] + msg, lastcol_ref[...]],
                           axis=1)
    influence = (jnp.dot(vodc, infw_ref[...],
                         preferred_element_type=jnp.float32)
                 + infb_ref[...])
    m = jnp.max(influence)
    w = jnp.exp(influence - m)
    att = w / jnp.sum(w)
    gr = jnp.sum(embed_ref[...] * att, axis=0,
                 keepdims=True)
    grc = jnp.transpose(gr)
    x = jnp.maximum(
        jnp.sum(w1_ref[...] * grc, axis=0, keepdims=True)
        + b1_ref[...], 0.0)
    xc = jnp.transpose(x)
    y = jnp.maximum(
        jnp.sum(w2_ref[...] * xc, axis=0, keepdims=True)
        + b2_ref[...], 0.0)
    yc = jnp.transpose(y)
    value_ref[...] = (jnp.sum(yc * wy_ref[...], axis=0, keepdims=True)
                      + by_ref[...])
    logits_ref[...] = (jnp.sum(yc * wx_ref[...], axis=0, keepdims=True)
                       + bx_ref[...]) / 10.0


def _stage3(vodes, partials, embed, lastcol, inf_W, inf_b,
            act_W1, act_b1, act_W2, act_b2, act_Wy, act_by, act_Wx, act_bx):
    return pl.pallas_call(
        _stage3_body,
        out_shape=[
            jax.ShapeDtypeStruct((1, 4), jnp.float32),
            jax.ShapeDtypeStruct((1, 1), jnp.float32),
        ],
    )(vodes, partials, embed, lastcol, inf_W, inf_b.reshape(1, 1),
      act_W1, act_b1.reshape(1, -1), act_W2, act_b2.reshape(1, -1),
      act_Wy, act_by.reshape(1, -1), act_Wx, act_bx.reshape(1, -1))


def kernel(nodes, senders, receivers, n_node, enc_W, enc_b, emb_W, emb_b,
           inf_W, inf_b, act_W1, act_b1, act_W2, act_b2, act_Wy, act_by,
           act_Wx, act_bx):
    vodes, embed = _stage1(nodes, enc_W, enc_b, emb_W, emb_b)
    partials = _stage2(vodes,
                       senders.astype(jnp.int32),
                       receivers.astype(jnp.int32))
    logits, value = _stage3(vodes, partials, embed, nodes[:, -1:],
                            inf_W, inf_b, act_W1, act_b1, act_W2, act_b2,
                            act_Wy, act_by, act_Wx, act_bx)
    return logits.reshape(4), value.reshape(1)


# speedup vs baseline: 1.1130x; 1.1130x over previous
import functools

import jax
import jax.numpy as jnp
from jax import lax
from jax.experimental import pallas as pl
from jax.experimental.pallas import tpu as pltpu
from jax.experimental.pallas import tpu_sc as plsc

_NC = 2
_NS = 16
_NW = _NC * _NS
_L = 16
_C = 125


def _stage1_body(nodes_ref, enc_W_ref, enc_b_ref, emb_W_ref, emb_b_ref,
                 vodes_ref, embed_ref, lastcol_ref):
    x = nodes_ref[...]
    vodes_ref[...] = jnp.maximum(
        jnp.dot(x, enc_W_ref[...], preferred_element_type=jnp.float32)
        + enc_b_ref[...], 0.0)
    embed_ref[...] = (
        jnp.dot(x, emb_W_ref[...], preferred_element_type=jnp.float32)
        + emb_b_ref[...])
    lastcol_ref[...] = x[:, -1:]


def _stage1(nodes, enc_W, enc_b, emb_W, emb_b):
    n, d = nodes.shape
    rows = 1000 if n % 1000 == 0 else n
    grid = n // rows
    de = enc_W.shape[1]
    dm = emb_W.shape[1]
    return pl.pallas_call(
        _stage1_body,
        grid=(grid,),
        in_specs=[
            pl.BlockSpec((rows, d), lambda i: (i, 0)),
            pl.BlockSpec((d, de), lambda i: (0, 0)),
            pl.BlockSpec((1, de), lambda i: (0, 0)),
            pl.BlockSpec((d, dm), lambda i: (0, 0)),
            pl.BlockSpec((1, dm), lambda i: (0, 0)),
        ],
        out_specs=[
            pl.BlockSpec((rows, de), lambda i: (i, 0)),
            pl.BlockSpec((rows, dm), lambda i: (i, 0)),
            pl.BlockSpec((rows, 1), lambda i: (i, 0)),
        ],
        out_shape=[
            jax.ShapeDtypeStruct((n, de), jnp.float32),
            jax.ShapeDtypeStruct((n, dm), jnp.float32),
            jax.ShapeDtypeStruct((n, 1), jnp.float32),
        ],
    )(nodes, enc_W, enc_b.reshape(1, de), emb_W, emb_b.reshape(1, dm))


def _sc_body(n, de, n_chunks, vodes_hbm, s2d_hbm, r2d_hbm, out_hbm,
             sbuf, rbuf, rows_v0, rows_v1, zbuf, acc_sh, vodes_sh,
             sem0, sem1):
    cid = lax.axis_index("c")
    sid = lax.axis_index("s")
    wid = sid * _NC + cid
    rows_lo = (n // _NS) // 8 * 8
    tail = n - _NS * rows_lo
    base = sid * rows_lo

    pltpu.sync_copy(s2d_hbm.at[pl.ds(wid * n_chunks, n_chunks)], sbuf)
    pltpu.sync_copy(r2d_hbm.at[pl.ds(wid * n_chunks, n_chunks)], rbuf)

    pltpu.sync_copy(vodes_hbm.at[pl.ds(base, rows_lo)],
                    vodes_sh.at[pl.ds(base, rows_lo)])

    @pl.when(sid == _NS - 1)
    def _():
        pltpu.sync_copy(vodes_hbm.at[pl.ds(_NS * rows_lo, tail)],
                        vodes_sh.at[pl.ds(_NS * rows_lo, tail)])

    def zero_body(i, _):
        zbuf[i, pl.ds(0, _L)] = jnp.zeros((_L,), jnp.float32)
        zbuf[i, pl.ds(_L, _L)] = jnp.zeros((_L,), jnp.float32)
        return 0

    lax.fori_loop(0, rows_lo, zero_body, 0)
    pltpu.sync_copy(zbuf, acc_sh.at[pl.ds(base, rows_lo)])

    @pl.when(sid == _NS - 1)
    def _():
        pltpu.sync_copy(zbuf.at[pl.ds(0, tail)],
                        acc_sh.at[pl.ds(_NS * rows_lo, tail)])

    plsc.subcore_barrier()

    n_pairs = n_chunks // 2

    def chunk_body(k, _):
        j0 = 2 * k
        j1 = j0 + 1
        pltpu.make_async_copy(vodes_sh.at[sbuf.at[j0]], rows_v0, sem0).wait()
        pltpu.async_copy(vodes_sh.at[sbuf.at[j1]], rows_v1, sem1)
        pltpu.sync_copy(rows_v0, acc_sh.at[rbuf.at[j0]], add=True)
        pltpu.make_async_copy(vodes_sh.at[sbuf.at[j1]], rows_v1, sem1).wait()
        pltpu.async_copy(vodes_sh.at[sbuf.at[j0 + 2]], rows_v0, sem0)
        pltpu.sync_copy(rows_v1, acc_sh.at[rbuf.at[j1]], add=True)
        return 0

    pltpu.async_copy(vodes_sh.at[sbuf.at[0]], rows_v0, sem0)
    lax.fori_loop(0, n_pairs - 1, chunk_body, 0)
    jl = n_chunks - 2
    pltpu.make_async_copy(vodes_sh.at[sbuf.at[jl]], rows_v0, sem0).wait()
    pltpu.async_copy(vodes_sh.at[sbuf.at[jl + 1]], rows_v1, sem1)
    pltpu.sync_copy(rows_v0, acc_sh.at[rbuf.at[jl]], add=True)
    pltpu.make_async_copy(vodes_sh.at[sbuf.at[jl + 1]], rows_v1, sem1).wait()
    pltpu.sync_copy(rows_v1, acc_sh.at[rbuf.at[jl + 1]], add=True)
    plsc.subcore_barrier()
    pltpu.sync_copy(acc_sh.at[pl.ds(base, rows_lo)],
                    out_hbm.at[cid, pl.ds(base, rows_lo)])

    @pl.when(sid == _NS - 1)
    def _():
        pltpu.sync_copy(acc_sh.at[pl.ds(_NS * rows_lo, tail)],
                        out_hbm.at[cid, pl.ds(_NS * rows_lo, tail)])


def _stage2(vodes, senders, receivers):
    n, de = vodes.shape
    e = senders.shape[0]
    n_chunks = e // (_NW * _C)
    s2d = senders.reshape(_NW * n_chunks, _C)
    r2d = receivers.reshape(_NW * n_chunks, _C)
    mesh = plsc.VectorSubcoreMesh(core_axis_name="c", subcore_axis_name="s")
    return pl.kernel(
        functools.partial(_sc_body, n, de, n_chunks),
        out_type=jax.ShapeDtypeStruct((_NC, n, de), jnp.float32),
        mesh=mesh,
        compiler_params=pltpu.CompilerParams(needs_layout_passes=False,
                                             use_tc_tiling_on_sc=False),
        scratch_types=[
            pltpu.VMEM((n_chunks, _C), jnp.int32),
            pltpu.VMEM((n_chunks, _C), jnp.int32),
            pltpu.VMEM((_C, de), jnp.float32),
            pltpu.VMEM((_C, de), jnp.float32),
            pltpu.VMEM(((n // _NS) // 8 * 8, de), jnp.float32),
            pltpu.VMEM_SHARED((n, de), jnp.float32),
            pltpu.VMEM_SHARED((n, de), jnp.float32),
            pltpu.SemaphoreType.DMA,
            pltpu.SemaphoreType.DMA,
        ],
    )(vodes, s2d, r2d)


def _stage3_body(vodes_ref, partials_ref, embed_ref, lastcol_ref, infw_ref,
                 infb_ref, w1_ref, b1_ref, w2_ref, b2_ref, wy_ref, by_ref,
                 wx_ref, bx_ref, logits_ref, value_ref):
    msg = partials_ref[0] + partials_ref[1]
    vodc = jnp.concatenate([vodes_ref[...] + msg, lastcol_ref[...]],
                           axis=1)
    influence = (jnp.dot(vodc, infw_ref[...],
                         preferred_element_type=jnp.float32)
                 + infb_ref[...])
    m = jnp.max(influence)
    w = jnp.exp(influence - m)
    att = w / jnp.sum(w)
    gr = jnp.sum(embed_ref[...] * att, axis=0,
                 keepdims=True)
    grc = jnp.transpose(gr)
    x = jnp.maximum(
        jnp.sum(w1_ref[...] * grc, axis=0, keepdims=True)
        + b1_ref[...], 0.0)
    xc = jnp.transpose(x)
    y = jnp.maximum(
        jnp.sum(w2_ref[...] * xc, axis=0, keepdims=True)
        + b2_ref[...], 0.0)
    yc = jnp.transpose(y)
    value_ref[...] = (jnp.sum(yc * wy_ref[...], axis=0, keepdims=True)
                      + by_ref[...])
    logits_ref[...] = (jnp.sum(yc * wx_ref[...], axis=0, keepdims=True)
                       + bx_ref[...]) / 10.0


def _stage3(vodes, partials, embed, lastcol, inf_W, inf_b,
            act_W1, act_b1, act_W2, act_b2, act_Wy, act_by, act_Wx, act_bx):
    return pl.pallas_call(
        _stage3_body,
        out_shape=[
            jax.ShapeDtypeStruct((1, 4), jnp.float32),
            jax.ShapeDtypeStruct((1, 1), jnp.float32),
        ],
    )(vodes, partials, embed, lastcol, inf_W, inf_b.reshape(1, 1),
      act_W1, act_b1.reshape(1, -1), act_W2, act_b2.reshape(1, -1),
      act_Wy, act_by.reshape(1, -1), act_Wx, act_bx.reshape(1, -1))


def kernel(nodes, senders, receivers, n_node, enc_W, enc_b, emb_W, emb_b,
           inf_W, inf_b, act_W1, act_b1, act_W2, act_b2, act_Wy, act_by,
           act_Wx, act_bx):
    vodes, embed, lastcol = _stage1(nodes, enc_W, enc_b, emb_W, emb_b)
    partials = _stage2(vodes,
                       senders.astype(jnp.int32),
                       receivers.astype(jnp.int32))
    logits, value = _stage3(vodes, partials, embed, lastcol,
                            inf_W, inf_b, act_W1, act_b1, act_W2, act_b2,
                            act_Wy, act_by, act_Wx, act_bx)
    return logits.reshape(4), value.reshape(1)
